# baseline (speedup 1.0000x reference)
import functools

import jax
import jax.numpy as jnp
from jax import lax
from jax.experimental import pallas as pl
from jax.experimental.pallas import tpu as pltpu
from jax.experimental.pallas import tpu_sc as plsc

N = 10000
D = 128
E = 320000
NC = 2
NS = 16
NW = NC * NS
K = 128
EPW = E // NW
CPWF = EPW // K
TAIL = EPW - CPWF * K
CPW = CPWF + 1
NQ2 = CPWF // 2
ROWS_SC = N
ZSEG = 624
ZLAST = ROWS_SC - (NS - 1) * ZSEG
WSEG = 624
WLAST = N - (NS - 1) * WSEG
L = 16

_mesh = plsc.VectorSubcoreMesh(core_axis_name="c", subcore_axis_name="s")


def _staged_copy(src_at, dst_at, seg_len, stage):
  sr = stage.shape[0]
  nfull = seg_len // sr
  for t in range(nfull):
    pltpu.sync_copy(src_at(t * sr, sr), stage)
    pltpu.sync_copy(stage, dst_at(t * sr, sr))
  rem = seg_len - nfull * sr
  if rem:
    pltpu.sync_copy(src_at(nfull * sr, rem), stage.at[pl.ds(0, rem)])
    pltpu.sync_copy(stage.at[pl.ds(0, rem)], dst_at(nfull * sr, rem))


def _zero_init(zsrc_hbm, sh, stage, base, seg_len):
  sr = stage.shape[0]
  pltpu.sync_copy(zsrc_hbm.at[pl.ds(0, sr)], stage)
  for t in range(seg_len // sr):
    pltpu.sync_copy(stage, sh.at[pl.ds(base + t * sr, sr)])
  rem = seg_len % sr
  if rem:
    pltpu.sync_copy(stage.at[pl.ds(0, rem)], sh.at[pl.ds(base + (seg_len // sr) * sr, rem)])


def _unpack_src(pk, j, dst_ref, n=K):
  for t in range(n // L):
    v = pk[j, pl.ds(t * L, L)]
    dst_ref[pl.ds(t * L, L)] = v & 0x3FFF


def _unpack_dst(pk, j, dst_ref, n=K):
  for t in range(n // L):
    v = pk[j, pl.ds(t * L, L)]
    dst_ref[pl.ds(t * L, L)] = lax.shift_right_logical(v, 14)


KS = 64
NSUB = CPWF * 2
NQ4 = NSUB // 4


def _unpack64(pk, row, half, is_ref, id_ref):
  for u in range(KS // L):
    v = pk[row, pl.ds(half * KS + u * L, L)]
    is_ref[pl.ds(u * L, L)] = v & 0x3FFF
    id_ref[pl.ds(u * L, L)] = lax.shift_right_logical(v, 14)


def _sc_acc_body(g_hbm, pk_hbm, zrow_hbm, acc_out,
                 pk, is0, is1, is2, is3, id0, id1, id2, id3, is_t, id_t,
                 rows0, rows1, rows2, rows3, acc_sh,
                 gs0, gs1, gs2, gs3, ss0, ss1, ss2, ss3):
  c = lax.axis_index("c")
  s = lax.axis_index("s")
  w = c * NS + s
  base = s * ZSEG
  rows = [rows0, rows1, rows2, rows3]
  isb = [is0, is1, is2, is3]
  idb = [id0, id1, id2, id3]
  gsem = [gs0, gs1, gs2, gs3]
  ssem = [ss0, ss1, ss2, ss3]

  @pl.when(s < NS - 1)
  def _():
    _zero_init(zrow_hbm, acc_sh, rows0, base, ZSEG)

  @pl.when(s == NS - 1)
  def _():
    _zero_init(zrow_hbm, acc_sh, rows0, base, ZLAST)

  pltpu.sync_copy(pk_hbm.at[w], pk)
  plsc.subcore_barrier()

  def gstart(b):
    pltpu.async_copy(g_hbm.at[isb[b]], rows[b], gsem[b])

  def gwait(b):
    pltpu.make_async_copy(g_hbm.at[isb[b]], rows[b], gsem[b]).wait()

  def sstart(b):
    pltpu.async_copy(rows[b], acc_sh.at[idb[b]], ssem[b], add=True)

  def swait(b):
    pltpu.make_async_copy(rows[b], acc_sh.at[idb[b]], ssem[b]).wait()

  for b in range(4):
    _unpack64(pk, b // 2, b % 2, isb[b], idb[b])
    gstart(b)

  def body(q, carry):
    for b in range(4):
      b2 = (b + 2) % 4
      def refill(bb=b2, row=2 * q + 1 + b // 2 if b < 2 else 2 * q + 2 + (b - 2) // 2,
                 half=b % 2):
        swait(bb)
        _unpack64(pk, row, half, isb[bb], idb[bb])
        gstart(bb)

      gwait(b)
      sstart(b)
      if b < 2:
        pl.when(q > 0)(refill)
      else:
        pl.when(q < NQ4 - 1)(refill)
    return carry

  lax.fori_loop(0, NQ4, body, 0)
  for b in range(4):
    swait(b)

  _unpack_src(pk, CPWF, is_t, n=TAIL)
  _unpack_dst(pk, CPWF, id_t, n=TAIL)
  pltpu.async_copy(g_hbm.at[is_t], rows0.at[pl.ds(0, TAIL)], gs0).wait()
  pltpu.sync_copy(rows0.at[pl.ds(0, TAIL)], acc_sh.at[id_t], add=True)

  plsc.subcore_barrier()


  @pl.when(s < NS - 1)
  def _():
    _staged_copy(lambda o, l: acc_sh.at[pl.ds(base + o, l)],
                 lambda o, l: acc_out.at[pl.ds(c * N + base + o, l)],
                 WSEG, rows0)

  @pl.when(s == NS - 1)
  def _():
    _staged_copy(lambda o, l: acc_sh.at[pl.ds(base + o, l)],
                 lambda o, l: acc_out.at[pl.ds(c * N + base + o, l)],
                 WLAST, rows0)


_sc_scatter = pl.kernel(
    _sc_acc_body,
    mesh=_mesh,
    out_type=jax.ShapeDtypeStruct((NC * N, D), jnp.float32),
    scratch_types=[
        pltpu.VMEM((CPW, K), jnp.int32),
        pltpu.VMEM((KS,), jnp.int32),
        pltpu.VMEM((KS,), jnp.int32),
        pltpu.VMEM((KS,), jnp.int32),
        pltpu.VMEM((KS,), jnp.int32),
        pltpu.VMEM((KS,), jnp.int32),
        pltpu.VMEM((KS,), jnp.int32),
        pltpu.VMEM((KS,), jnp.int32),
        pltpu.VMEM((KS,), jnp.int32),
        pltpu.VMEM((TAIL,), jnp.int32),
        pltpu.VMEM((TAIL,), jnp.int32),
        pltpu.VMEM((KS, D), jnp.float32),
        pltpu.VMEM((KS, D), jnp.float32),
        pltpu.VMEM((KS, D), jnp.float32),
        pltpu.VMEM((KS, D), jnp.float32),
        pltpu.VMEM_SHARED((ROWS_SC, D), jnp.float32),
        pltpu.SemaphoreType.DMA,
        pltpu.SemaphoreType.DMA,
        pltpu.SemaphoreType.DMA,
        pltpu.SemaphoreType.DMA,
        pltpu.SemaphoreType.DMA,
        pltpu.SemaphoreType.DMA,
        pltpu.SemaphoreType.DMA,
        pltpu.SemaphoreType.DMA,
    ],
)


def _sc_cnt_body(pk_hbm, zrow_hbm, ones_hbm, cnt_out,
                 pk, id0, id1, id_t, ones_v, cbuf, cnt_sh, ss0, ss1):
  c = lax.axis_index("c")
  s = lax.axis_index("s")
  w = c * NS + s
  base = s * ZSEG
  idb = [id0, id1]
  ssem = [ss0, ss1]

  @pl.when(s < NS - 1)
  def _():
    _zero_init(zrow_hbm, cnt_sh, cbuf, base, ZSEG)

  @pl.when(s == NS - 1)
  def _():
    _zero_init(zrow_hbm, cnt_sh, cbuf, base, ZLAST)

  pltpu.sync_copy(ones_hbm, ones_v)
  pltpu.sync_copy(pk_hbm.at[w], pk)
  plsc.subcore_barrier()

  def sstart(b):
    pltpu.async_copy(ones_v, cnt_sh.at[idb[b]], ssem[b], add=True)

  def swait(b):
    pltpu.make_async_copy(ones_v, cnt_sh.at[idb[b]], ssem[b]).wait()

  def body(q, carry):
    j0 = 2 * q

    @pl.when(q > 0)
    def _():
      swait(0)
    _unpack_dst(pk, j0, id0)
    sstart(0)

    @pl.when(q > 0)
    def _():
      swait(1)
    _unpack_dst(pk, j0 + 1, id1)
    sstart(1)
    return carry

  lax.fori_loop(0, CPWF // 2, body, 0)
  swait(0)
  swait(1)

  _unpack_dst(pk, CPWF, id_t, n=TAIL)
  pltpu.sync_copy(ones_v.at[pl.ds(0, TAIL)], cnt_sh.at[id_t], add=True)

  plsc.subcore_barrier()

  @pl.when(s < NS - 1)
  def _():
    _staged_copy(lambda o, l: cnt_sh.at[pl.ds(base + o, l)],
                 lambda o, l: cnt_out.at[pl.ds(c * N + base + o, l)],
                 WSEG, cbuf)

  @pl.when(s == NS - 1)
  def _():
    _staged_copy(lambda o, l: cnt_sh.at[pl.ds(base + o, l)],
                 lambda o, l: cnt_out.at[pl.ds(c * N + base + o, l)],
                 WLAST, cbuf)


_sc_cnt = pl.kernel(
    _sc_cnt_body,
    mesh=_mesh,
    out_type=jax.ShapeDtypeStruct((NC * N, D), jnp.float32),
    scratch_types=[
        pltpu.VMEM((CPW, K), jnp.int32),
        pltpu.VMEM((K,), jnp.int32),
        pltpu.VMEM((K,), jnp.int32),
        pltpu.VMEM((TAIL,), jnp.int32),
        pltpu.VMEM((K, D), jnp.float32),
        pltpu.VMEM((K, D), jnp.float32),
        pltpu.VMEM_SHARED((ROWS_SC, D), jnp.float32),
        pltpu.SemaphoreType.DMA,
        pltpu.SemaphoreType.DMA,
    ],
)


def _tc_pre_body(x_ref, wl_ref, wr_ref, b_ref, g_ref, r_ref):
  h = x_ref[...]
  g_ref[...] = jnp.dot(h, wl_ref[...], preferred_element_type=jnp.float32)
  r_ref[...] = jnp.dot(h, wr_ref[...], preferred_element_type=jnp.float32) + b_ref[...]


_tc_pre = pl.pallas_call(
    _tc_pre_body,
    out_shape=[jax.ShapeDtypeStruct((N, D), jnp.float32),
               jax.ShapeDtypeStruct((N, D), jnp.float32)],
)


def _tc_mid_body(acc_ref, cnt_ref, rp_ref, wl_ref, wr_ref, b_ref,
                 g_ref, r_ref):
  acc = acc_ref[0] + acc_ref[1]
  cnt = cnt_ref[0] + cnt_ref[1]
  inv = 1.0 / jnp.maximum(cnt, 1.0)
  h = jnp.maximum(acc * inv + rp_ref[...], 0.0)
  g_ref[...] = jnp.dot(h, wl_ref[...], preferred_element_type=jnp.float32)
  r_ref[...] = jnp.dot(h, wr_ref[...], preferred_element_type=jnp.float32) + b_ref[...]


_tc_mid = pl.pallas_call(
    _tc_mid_body,
    out_shape=[jax.ShapeDtypeStruct((N, D), jnp.float32),
               jax.ShapeDtypeStruct((N, D), jnp.float32)],
)


def _tc_post_body(acc_ref, cnt_ref, rp_ref, out_ref):
  acc = acc_ref[0] + acc_ref[1]
  cnt = cnt_ref[0] + cnt_ref[1]
  inv = 1.0 / jnp.maximum(cnt, 1.0)
  out_ref[...] = acc * inv + rp_ref[...]


_tc_post = pl.pallas_call(
    _tc_post_body,
    out_shape=jax.ShapeDtypeStruct((N, D), jnp.float32),
)


def kernel(x, edge_index, Wl0, Wr0, b0, Wl1, Wr1, b1, Wl2, Wr2, b2):
  src = edge_index[0].astype(jnp.int32)
  dst = edge_index[1].astype(jnp.int32)
  ppw = CPW * K - EPW
  src_p = jnp.concatenate(
      [src.reshape(NW, EPW), jnp.zeros((NW, ppw), jnp.int32)], axis=1)
  dst_p = jnp.concatenate(
      [dst.reshape(NW, EPW), jnp.zeros((NW, ppw), jnp.int32)], axis=1)
  pk = (src_p | (dst_p << 14)).reshape(NW, CPW, K)
  zrow = jnp.zeros((K, D), jnp.float32)
  onesK = jnp.ones((K, D), jnp.float32)

  cnt = _sc_cnt(pk, zrow, onesK).reshape(NC, N, D)
  g0, r0 = _tc_pre(x, Wl0.T, Wr0.T, b0.reshape(1, D))
  acc0 = _sc_scatter(g0, pk, zrow).reshape(NC, N, D)
  g1, r1 = _tc_mid(acc0, cnt, r0, Wl1.T, Wr1.T, b1.reshape(1, D))
  acc1 = _sc_scatter(g1, pk, zrow).reshape(NC, N, D)
  g2, r2 = _tc_mid(acc1, cnt, r1, Wl2.T, Wr2.T, b2.reshape(1, D))
  acc2 = _sc_scatter(g2, pk, zrow).reshape(NC, N, D)
  return _tc_post(acc2, cnt, r2)


# speedup vs baseline: 1.0452x; 1.0452x over previous
import functools

import jax
import jax.numpy as jnp
from jax import lax
from jax.experimental import pallas as pl
from jax.experimental.pallas import tpu as pltpu
from jax.experimental.pallas import tpu_sc as plsc

N = 10000
D = 128
E = 320000
NC = 2
NS = 16
NW = NC * NS
K = 128
EPW = E // NW
CPWF = EPW // K
TAIL = EPW - CPWF * K
CPW = CPWF + 1
NQ2 = CPWF // 2
ROWS_SC = N
ZSEG = 624
ZLAST = ROWS_SC - (NS - 1) * ZSEG
WSEG = 624
WLAST = N - (NS - 1) * WSEG
L = 16

_mesh = plsc.VectorSubcoreMesh(core_axis_name="c", subcore_axis_name="s")


def _staged_copy(src_at, dst_at, seg_len, stage):
  sr = stage.shape[0]
  nfull = seg_len // sr
  for t in range(nfull):
    pltpu.sync_copy(src_at(t * sr, sr), stage)
    pltpu.sync_copy(stage, dst_at(t * sr, sr))
  rem = seg_len - nfull * sr
  if rem:
    pltpu.sync_copy(src_at(nfull * sr, rem), stage.at[pl.ds(0, rem)])
    pltpu.sync_copy(stage.at[pl.ds(0, rem)], dst_at(nfull * sr, rem))


def _zero_init(zsrc_hbm, sh, stage, base, seg_len):
  sr = stage.shape[0]
  pltpu.sync_copy(zsrc_hbm.at[pl.ds(0, sr)], stage)
  for t in range(seg_len // sr):
    pltpu.sync_copy(stage, sh.at[pl.ds(base + t * sr, sr)])
  rem = seg_len % sr
  if rem:
    pltpu.sync_copy(stage.at[pl.ds(0, rem)], sh.at[pl.ds(base + (seg_len // sr) * sr, rem)])


def _unpack_src(pk, j, dst_ref, n=K):
  for t in range(n // L):
    v = pk[j, pl.ds(t * L, L)]
    dst_ref[pl.ds(t * L, L)] = v & 0x3FFF


def _unpack_dst(pk, j, dst_ref, n=K):
  for t in range(n // L):
    v = pk[j, pl.ds(t * L, L)]
    dst_ref[pl.ds(t * L, L)] = lax.shift_right_logical(v, 14)


KS = 32
RB = 8
SPC = K // KS
NSUB = CPWF * SPC
NQR = NSUB // RB


def _unpackKS(pk, row, part, is_ref, id_ref):
  for u in range(KS // L):
    v = pk[row, pl.ds(part * KS + u * L, L)]
    is_ref[pl.ds(u * L, L)] = v & 0x3FFF
    id_ref[pl.ds(u * L, L)] = lax.shift_right_logical(v, 14)


def _sc_acc_body(g_hbm, pk_hbm, zrow_hbm, acc_out, pk, *rest):
  isb = list(rest[0:RB])
  idb = list(rest[RB:2 * RB])
  is_t, id_t = rest[2 * RB], rest[2 * RB + 1]
  rows = list(rest[2 * RB + 2:3 * RB + 2])
  acc_sh = rest[3 * RB + 2]
  gsem = list(rest[3 * RB + 3:4 * RB + 3])
  ssem = list(rest[4 * RB + 3:5 * RB + 3])

  c = lax.axis_index("c")
  s = lax.axis_index("s")
  w = c * NS + s
  base = s * ZSEG

  @pl.when(s < NS - 1)
  def _():
    _zero_init(zrow_hbm, acc_sh, rows[0], base, ZSEG)

  @pl.when(s == NS - 1)
  def _():
    _zero_init(zrow_hbm, acc_sh, rows[0], base, ZLAST)

  pltpu.sync_copy(pk_hbm.at[w], pk)
  plsc.subcore_barrier()

  def gstart(b):
    pltpu.async_copy(g_hbm.at[isb[b]], rows[b], gsem[b])

  def gwait(b):
    pltpu.make_async_copy(g_hbm.at[isb[b]], rows[b], gsem[b]).wait()

  def sstart(b):
    pltpu.async_copy(rows[b], acc_sh.at[idb[b]], ssem[b], add=True)

  def swait(b):
    pltpu.make_async_copy(rows[b], acc_sh.at[idb[b]], ssem[b]).wait()

  for b in range(RB):
    _unpackKS(pk, b // SPC, b % SPC, isb[b], idb[b])
    gstart(b)

  HB = RB // 2

  def body(q, carry):
    for b in range(RB):
      b2 = (b + HB) % RB
      def refill(bb=b2,
                 row=(2 * q + 1 + b // SPC) if b < HB else (2 * q + 2 + (b - HB) // SPC),
                 part=b % SPC):
        swait(bb)
        _unpackKS(pk, row, part, isb[bb], idb[bb])
        gstart(bb)

      gwait(b)
      sstart(b)
      if b < HB:
        pl.when(q > 0)(refill)
      else:
        pl.when(q < NQR - 1)(refill)
    return carry

  lax.fori_loop(0, NQR, body, 0)
  for b in range(RB):
    swait(b)

  _unpack_src(pk, CPWF, is_t, n=TAIL)
  _unpack_dst(pk, CPWF, id_t, n=TAIL)
  pltpu.async_copy(g_hbm.at[is_t], rows[0].at[pl.ds(0, TAIL)], gsem[0]).wait()
  pltpu.sync_copy(rows[0].at[pl.ds(0, TAIL)], acc_sh.at[id_t], add=True)

  plsc.subcore_barrier()


  @pl.when(s < NS - 1)
  def _():
    _staged_copy(lambda o, l: acc_sh.at[pl.ds(base + o, l)],
                 lambda o, l: acc_out.at[pl.ds(c * N + base + o, l)],
                 WSEG, rows[0])

  @pl.when(s == NS - 1)
  def _():
    _staged_copy(lambda o, l: acc_sh.at[pl.ds(base + o, l)],
                 lambda o, l: acc_out.at[pl.ds(c * N + base + o, l)],
                 WLAST, rows[0])


_sc_scatter = pl.kernel(
    _sc_acc_body,
    mesh=_mesh,
    out_type=jax.ShapeDtypeStruct((NC * N, D), jnp.float32),
    scratch_types=(
        [pltpu.VMEM((CPW, K), jnp.int32)]
        + [pltpu.VMEM((KS,), jnp.int32) for _ in range(2 * RB)]
        + [pltpu.VMEM((TAIL,), jnp.int32) for _ in range(2)]
        + [pltpu.VMEM((KS, D), jnp.float32) for _ in range(RB)]
        + [pltpu.VMEM_SHARED((ROWS_SC, D), jnp.float32)]
        + [pltpu.SemaphoreType.DMA for _ in range(2 * RB)]
    ),
)


def _sc_cnt_body(pk_hbm, zrow_hbm, ones_hbm, cnt_out,
                 pk, id0, id1, id_t, ones_v, cbuf, cnt_sh, ss0, ss1):
  c = lax.axis_index("c")
  s = lax.axis_index("s")
  w = c * NS + s
  base = s * ZSEG
  idb = [id0, id1]
  ssem = [ss0, ss1]

  @pl.when(s < NS - 1)
  def _():
    _zero_init(zrow_hbm, cnt_sh, cbuf, base, ZSEG)

  @pl.when(s == NS - 1)
  def _():
    _zero_init(zrow_hbm, cnt_sh, cbuf, base, ZLAST)

  pltpu.sync_copy(ones_hbm, ones_v)
  pltpu.sync_copy(pk_hbm.at[w], pk)
  plsc.subcore_barrier()

  def sstart(b):
    pltpu.async_copy(ones_v, cnt_sh.at[idb[b]], ssem[b], add=True)

  def swait(b):
    pltpu.make_async_copy(ones_v, cnt_sh.at[idb[b]], ssem[b]).wait()

  def body(q, carry):
    j0 = 2 * q

    @pl.when(q > 0)
    def _():
      swait(0)
    _unpack_dst(pk, j0, id0)
    sstart(0)

    @pl.when(q > 0)
    def _():
      swait(1)
    _unpack_dst(pk, j0 + 1, id1)
    sstart(1)
    return carry

  lax.fori_loop(0, CPWF // 2, body, 0)
  swait(0)
  swait(1)

  _unpack_dst(pk, CPWF, id_t, n=TAIL)
  pltpu.sync_copy(ones_v.at[pl.ds(0, TAIL)], cnt_sh.at[id_t], add=True)

  plsc.subcore_barrier()

  @pl.when(s < NS - 1)
  def _():
    _staged_copy(lambda o, l: cnt_sh.at[pl.ds(base + o, l)],
                 lambda o, l: cnt_out.at[pl.ds(c * N + base + o, l)],
                 WSEG, cbuf)

  @pl.when(s == NS - 1)
  def _():
    _staged_copy(lambda o, l: cnt_sh.at[pl.ds(base + o, l)],
                 lambda o, l: cnt_out.at[pl.ds(c * N + base + o, l)],
                 WLAST, cbuf)


_sc_cnt = pl.kernel(
    _sc_cnt_body,
    mesh=_mesh,
    out_type=jax.ShapeDtypeStruct((NC * N, D), jnp.float32),
    scratch_types=[
        pltpu.VMEM((CPW, K), jnp.int32),
        pltpu.VMEM((K,), jnp.int32),
        pltpu.VMEM((K,), jnp.int32),
        pltpu.VMEM((TAIL,), jnp.int32),
        pltpu.VMEM((K, D), jnp.float32),
        pltpu.VMEM((K, D), jnp.float32),
        pltpu.VMEM_SHARED((ROWS_SC, D), jnp.float32),
        pltpu.SemaphoreType.DMA,
        pltpu.SemaphoreType.DMA,
    ],
)


def _tc_pre_body(x_ref, wl_ref, wr_ref, b_ref, g_ref, r_ref):
  h = x_ref[...]
  g_ref[...] = jnp.dot(h, wl_ref[...], preferred_element_type=jnp.float32)
  r_ref[...] = jnp.dot(h, wr_ref[...], preferred_element_type=jnp.float32) + b_ref[...]


_tc_pre = pl.pallas_call(
    _tc_pre_body,
    out_shape=[jax.ShapeDtypeStruct((N, D), jnp.float32),
               jax.ShapeDtypeStruct((N, D), jnp.float32)],
)


def _tc_mid_body(acc_ref, cnt_ref, rp_ref, wl_ref, wr_ref, b_ref,
                 g_ref, r_ref):
  acc = acc_ref[0] + acc_ref[1]
  cnt = cnt_ref[0] + cnt_ref[1]
  inv = 1.0 / jnp.maximum(cnt, 1.0)
  h = jnp.maximum(acc * inv + rp_ref[...], 0.0)
  g_ref[...] = jnp.dot(h, wl_ref[...], preferred_element_type=jnp.float32)
  r_ref[...] = jnp.dot(h, wr_ref[...], preferred_element_type=jnp.float32) + b_ref[...]


_tc_mid = pl.pallas_call(
    _tc_mid_body,
    out_shape=[jax.ShapeDtypeStruct((N, D), jnp.float32),
               jax.ShapeDtypeStruct((N, D), jnp.float32)],
)


def _tc_post_body(acc_ref, cnt_ref, rp_ref, out_ref):
  acc = acc_ref[0] + acc_ref[1]
  cnt = cnt_ref[0] + cnt_ref[1]
  inv = 1.0 / jnp.maximum(cnt, 1.0)
  out_ref[...] = acc * inv + rp_ref[...]


_tc_post = pl.pallas_call(
    _tc_post_body,
    out_shape=jax.ShapeDtypeStruct((N, D), jnp.float32),
)


def kernel(x, edge_index, Wl0, Wr0, b0, Wl1, Wr1, b1, Wl2, Wr2, b2):
  src = edge_index[0].astype(jnp.int32)
  dst = edge_index[1].astype(jnp.int32)
  ppw = CPW * K - EPW
  src_p = jnp.concatenate(
      [src.reshape(NW, EPW), jnp.zeros((NW, ppw), jnp.int32)], axis=1)
  dst_p = jnp.concatenate(
      [dst.reshape(NW, EPW), jnp.zeros((NW, ppw), jnp.int32)], axis=1)
  pk = (src_p | (dst_p << 14)).reshape(NW, CPW, K)
  zrow = jnp.zeros((K, D), jnp.float32)
  onesK = jnp.ones((K, D), jnp.float32)

  cnt = _sc_cnt(pk, zrow, onesK).reshape(NC, N, D)
  g0, r0 = _tc_pre(x, Wl0.T, Wr0.T, b0.reshape(1, D))
  acc0 = _sc_scatter(g0, pk, zrow).reshape(NC, N, D)
  g1, r1 = _tc_mid(acc0, cnt, r0, Wl1.T, Wr1.T, b1.reshape(1, D))
  acc1 = _sc_scatter(g1, pk, zrow).reshape(NC, N, D)
  g2, r2 = _tc_mid(acc1, cnt, r1, Wl2.T, Wr2.T, b2.reshape(1, D))
  acc2 = _sc_scatter(g2, pk, zrow).reshape(NC, N, D)
  return _tc_post(acc2, cnt, r2)


# speedup vs baseline: 1.0580x; 1.0123x over previous
import functools

import jax
import jax.numpy as jnp
from jax import lax
from jax.experimental import pallas as pl
from jax.experimental.pallas import tpu as pltpu
from jax.experimental.pallas import tpu_sc as plsc

N = 10000
D = 128
E = 320000
NC = 2
NS = 16
NW = NC * NS
K = 128
EPW = E // NW
CPWF = EPW // K
TAIL = EPW - CPWF * K
CPW = CPWF + 1
NQ2 = CPWF // 2
ROWS_SC = N
ZSEG = 624
ZLAST = ROWS_SC - (NS - 1) * ZSEG
WSEG = 624
WLAST = N - (NS - 1) * WSEG
L = 16

_mesh = plsc.VectorSubcoreMesh(core_axis_name="c", subcore_axis_name="s")


def _staged_copy(src_at, dst_at, seg_len, stage):
  sr = stage.shape[0]
  nfull = seg_len // sr
  for t in range(nfull):
    pltpu.sync_copy(src_at(t * sr, sr), stage)
    pltpu.sync_copy(stage, dst_at(t * sr, sr))
  rem = seg_len - nfull * sr
  if rem:
    pltpu.sync_copy(src_at(nfull * sr, rem), stage.at[pl.ds(0, rem)])
    pltpu.sync_copy(stage.at[pl.ds(0, rem)], dst_at(nfull * sr, rem))


def _zero_init(zsrc_hbm, sh, stage, base, seg_len):
  sr = stage.shape[0]
  pltpu.sync_copy(zsrc_hbm.at[pl.ds(0, sr)], stage)
  for t in range(seg_len // sr):
    pltpu.sync_copy(stage, sh.at[pl.ds(base + t * sr, sr)])
  rem = seg_len % sr
  if rem:
    pltpu.sync_copy(stage.at[pl.ds(0, rem)], sh.at[pl.ds(base + (seg_len // sr) * sr, rem)])


def _unpack_src(pk, j, dst_ref, n=K):
  for t in range(n // L):
    v = pk[j, pl.ds(t * L, L)]
    dst_ref[pl.ds(t * L, L)] = v & 0x3FFF


def _unpack_dst(pk, j, dst_ref, n=K):
  for t in range(n // L):
    v = pk[j, pl.ds(t * L, L)]
    dst_ref[pl.ds(t * L, L)] = lax.shift_right_logical(v, 14)


KS = 32
RB = 8
SPC = K // KS
NSUB = CPWF * SPC
NQR = NSUB // RB


def _unpackKS(pk, row, part, is_ref, id_ref):
  for u in range(KS // L):
    v = pk[row, pl.ds(part * KS + u * L, L)]
    is_ref[pl.ds(u * L, L)] = v & 0x3FFF
    id_ref[pl.ds(u * L, L)] = lax.shift_right_logical(v, 14)


def _sc_acc_body(g_hbm, pk_hbm, zrow_hbm, acc_out, pk, *rest):
  isb = list(rest[0:RB])
  idb = list(rest[RB:2 * RB])
  is_t, id_t = rest[2 * RB], rest[2 * RB + 1]
  rows = list(rest[2 * RB + 2:3 * RB + 2])
  acc_sh = rest[3 * RB + 2]
  gsem = list(rest[3 * RB + 3:4 * RB + 3])
  ssem = list(rest[4 * RB + 3:5 * RB + 3])

  c = lax.axis_index("c")
  s = lax.axis_index("s")
  w = c * NS + s
  base = s * ZSEG

  @pl.when(s < NS - 1)
  def _():
    _zero_init(zrow_hbm, acc_sh, rows[0], base, ZSEG)

  @pl.when(s == NS - 1)
  def _():
    _zero_init(zrow_hbm, acc_sh, rows[0], base, ZLAST)

  pltpu.sync_copy(pk_hbm.at[w], pk)
  plsc.subcore_barrier()

  def gstart(b):
    pltpu.async_copy(g_hbm.at[isb[b]], rows[b], gsem[b])

  def gwait(b):
    pltpu.make_async_copy(g_hbm.at[isb[b]], rows[b], gsem[b]).wait()

  def sstart(b):
    pltpu.async_copy(rows[b], acc_sh.at[idb[b]], ssem[b], add=True)

  def swait(b):
    pltpu.make_async_copy(rows[b], acc_sh.at[idb[b]], ssem[b]).wait()

  for b in range(RB):
    _unpackKS(pk, b // SPC, b % SPC, isb[b], idb[b])
    gstart(b)

  HB = RB // 2

  def body(q, carry):
    for b in range(RB):
      b2 = (b + HB) % RB
      def refill(bb=b2,
                 row=(2 * q + 1 + b // SPC) if b < HB else (2 * q + 2 + (b - HB) // SPC),
                 part=b % SPC):
        swait(bb)
        _unpackKS(pk, row, part, isb[bb], idb[bb])
        gstart(bb)

      gwait(b)
      sstart(b)
      if b < HB:
        pl.when(q > 0)(refill)
      else:
        pl.when(q < NQR - 1)(refill)
    return carry

  lax.fori_loop(0, NQR, body, 0)
  for b in range(RB):
    swait(b)

  def wb_async(seg_len):
    segs = [(o, min(KS, seg_len - o)) for o in range(0, seg_len, KS)]
    for i, (o, l) in enumerate(segs):
      b = i % RB
      if i >= RB:
        po, pl_ = segs[i - RB]
        pltpu.make_async_copy(
            rows[b].at[pl.ds(0, pl_)],
            acc_out.at[pl.ds(c * N + base + po, pl_)], ssem[b]).wait()
      pltpu.sync_copy(acc_sh.at[pl.ds(base + o, l)], rows[b].at[pl.ds(0, l)])
      pltpu.async_copy(rows[b].at[pl.ds(0, l)],
                       acc_out.at[pl.ds(c * N + base + o, l)], ssem[b])
    ntail = min(RB, len(segs))
    for i in range(len(segs) - ntail, len(segs)):
      o, l = segs[i]
      pltpu.make_async_copy(rows[i % RB].at[pl.ds(0, l)],
                            acc_out.at[pl.ds(c * N + base + o, l)],
                            ssem[i % RB]).wait()

  _unpack_src(pk, CPWF, is_t, n=TAIL)
  _unpack_dst(pk, CPWF, id_t, n=TAIL)
  pltpu.async_copy(g_hbm.at[is_t], rows[0].at[pl.ds(0, TAIL)], gsem[0]).wait()
  pltpu.sync_copy(rows[0].at[pl.ds(0, TAIL)], acc_sh.at[id_t], add=True)

  plsc.subcore_barrier()


  @pl.when(s < NS - 1)
  def _():
    wb_async(WSEG)

  @pl.when(s == NS - 1)
  def _():
    wb_async(WLAST)


_sc_scatter = pl.kernel(
    _sc_acc_body,
    mesh=_mesh,
    out_type=jax.ShapeDtypeStruct((NC * N, D), jnp.float32),
    scratch_types=(
        [pltpu.VMEM((CPW, K), jnp.int32)]
        + [pltpu.VMEM((KS,), jnp.int32) for _ in range(2 * RB)]
        + [pltpu.VMEM((TAIL,), jnp.int32) for _ in range(2)]
        + [pltpu.VMEM((KS, D), jnp.float32) for _ in range(RB)]
        + [pltpu.VMEM_SHARED((ROWS_SC, D), jnp.float32)]
        + [pltpu.SemaphoreType.DMA for _ in range(2 * RB)]
    ),
)


def _sc_cnt_body(pk_hbm, zrow_hbm, ones_hbm, cnt_out,
                 pk, id0, id1, id_t, ones_v, cbuf, cnt_sh, ss0, ss1):
  c = lax.axis_index("c")
  s = lax.axis_index("s")
  w = c * NS + s
  base = s * ZSEG
  idb = [id0, id1]
  ssem = [ss0, ss1]

  @pl.when(s < NS - 1)
  def _():
    _zero_init(zrow_hbm, cnt_sh, cbuf, base, ZSEG)

  @pl.when(s == NS - 1)
  def _():
    _zero_init(zrow_hbm, cnt_sh, cbuf, base, ZLAST)

  pltpu.sync_copy(ones_hbm, ones_v)
  pltpu.sync_copy(pk_hbm.at[w], pk)
  plsc.subcore_barrier()

  def sstart(b):
    pltpu.async_copy(ones_v, cnt_sh.at[idb[b]], ssem[b], add=True)

  def swait(b):
    pltpu.make_async_copy(ones_v, cnt_sh.at[idb[b]], ssem[b]).wait()

  def body(q, carry):
    j0 = 2 * q

    @pl.when(q > 0)
    def _():
      swait(0)
    _unpack_dst(pk, j0, id0)
    sstart(0)

    @pl.when(q > 0)
    def _():
      swait(1)
    _unpack_dst(pk, j0 + 1, id1)
    sstart(1)
    return carry

  lax.fori_loop(0, CPWF // 2, body, 0)
  swait(0)
  swait(1)

  _unpack_dst(pk, CPWF, id_t, n=TAIL)
  pltpu.sync_copy(ones_v.at[pl.ds(0, TAIL)], cnt_sh.at[id_t], add=True)

  plsc.subcore_barrier()

  @pl.when(s < NS - 1)
  def _():
    _staged_copy(lambda o, l: cnt_sh.at[pl.ds(base + o, l)],
                 lambda o, l: cnt_out.at[pl.ds(c * N + base + o, l)],
                 WSEG, cbuf)

  @pl.when(s == NS - 1)
  def _():
    _staged_copy(lambda o, l: cnt_sh.at[pl.ds(base + o, l)],
                 lambda o, l: cnt_out.at[pl.ds(c * N + base + o, l)],
                 WLAST, cbuf)


_sc_cnt = pl.kernel(
    _sc_cnt_body,
    mesh=_mesh,
    out_type=jax.ShapeDtypeStruct((NC * N, D), jnp.float32),
    scratch_types=[
        pltpu.VMEM((CPW, K), jnp.int32),
        pltpu.VMEM((K,), jnp.int32),
        pltpu.VMEM((K,), jnp.int32),
        pltpu.VMEM((TAIL,), jnp.int32),
        pltpu.VMEM((K, D), jnp.float32),
        pltpu.VMEM((K, D), jnp.float32),
        pltpu.VMEM_SHARED((ROWS_SC, D), jnp.float32),
        pltpu.SemaphoreType.DMA,
        pltpu.SemaphoreType.DMA,
    ],
)


def _tc_pre_body(x_ref, wl_ref, wr_ref, b_ref, g_ref, r_ref):
  h = x_ref[...]
  g_ref[...] = jnp.dot(h, wl_ref[...], preferred_element_type=jnp.float32)
  r_ref[...] = jnp.dot(h, wr_ref[...], preferred_element_type=jnp.float32) + b_ref[...]


_tc_pre = pl.pallas_call(
    _tc_pre_body,
    out_shape=[jax.ShapeDtypeStruct((N, D), jnp.float32),
               jax.ShapeDtypeStruct((N, D), jnp.float32)],
)


def _tc_mid_body(acc_ref, cnt_ref, rp_ref, wl_ref, wr_ref, b_ref,
                 g_ref, r_ref):
  acc = acc_ref[0] + acc_ref[1]
  cnt = cnt_ref[0] + cnt_ref[1]
  inv = 1.0 / jnp.maximum(cnt, 1.0)
  h = jnp.maximum(acc * inv + rp_ref[...], 0.0)
  g_ref[...] = jnp.dot(h, wl_ref[...], preferred_element_type=jnp.float32)
  r_ref[...] = jnp.dot(h, wr_ref[...], preferred_element_type=jnp.float32) + b_ref[...]


_tc_mid = pl.pallas_call(
    _tc_mid_body,
    out_shape=[jax.ShapeDtypeStruct((N, D), jnp.float32),
               jax.ShapeDtypeStruct((N, D), jnp.float32)],
)


def _tc_post_body(acc_ref, cnt_ref, rp_ref, out_ref):
  acc = acc_ref[0] + acc_ref[1]
  cnt = cnt_ref[0] + cnt_ref[1]
  inv = 1.0 / jnp.maximum(cnt, 1.0)
  out_ref[...] = acc * inv + rp_ref[...]


_tc_post = pl.pallas_call(
    _tc_post_body,
    out_shape=jax.ShapeDtypeStruct((N, D), jnp.float32),
)


def kernel(x, edge_index, Wl0, Wr0, b0, Wl1, Wr1, b1, Wl2, Wr2, b2):
  src = edge_index[0].astype(jnp.int32)
  dst = edge_index[1].astype(jnp.int32)
  ppw = CPW * K - EPW
  src_p = jnp.concatenate(
      [src.reshape(NW, EPW), jnp.zeros((NW, ppw), jnp.int32)], axis=1)
  dst_p = jnp.concatenate(
      [dst.reshape(NW, EPW), jnp.zeros((NW, ppw), jnp.int32)], axis=1)
  pk = (src_p | (dst_p << 14)).reshape(NW, CPW, K)
  zrow = jnp.zeros((K, D), jnp.float32)
  onesK = jnp.ones((K, D), jnp.float32)

  cnt = _sc_cnt(pk, zrow, onesK).reshape(NC, N, D)
  g0, r0 = _tc_pre(x, Wl0.T, Wr0.T, b0.reshape(1, D))
  acc0 = _sc_scatter(g0, pk, zrow).reshape(NC, N, D)
  g1, r1 = _tc_mid(acc0, cnt, r0, Wl1.T, Wr1.T, b1.reshape(1, D))
  acc1 = _sc_scatter(g1, pk, zrow).reshape(NC, N, D)
  g2, r2 = _tc_mid(acc1, cnt, r1, Wl2.T, Wr2.T, b2.reshape(1, D))
  acc2 = _sc_scatter(g2, pk, zrow).reshape(NC, N, D)
  return _tc_post(acc2, cnt, r2)


# speedup vs baseline: 1.0732x; 1.0144x over previous
import functools

import jax
import jax.numpy as jnp
from jax import lax
from jax.experimental import pallas as pl
from jax.experimental.pallas import tpu as pltpu
from jax.experimental.pallas import tpu_sc as plsc

N = 10000
D = 128
E = 320000
NC = 2
NS = 16
NW = NC * NS
K = 128
EPW = E // NW
CPWF = EPW // K
TAIL = EPW - CPWF * K
CPW = CPWF + 1
NQ2 = CPWF // 2
ROWS_SC = N
ZSEG = 624
ZLAST = ROWS_SC - (NS - 1) * ZSEG
WSEG = 624
WLAST = N - (NS - 1) * WSEG
L = 16

_mesh = plsc.VectorSubcoreMesh(core_axis_name="c", subcore_axis_name="s")


def _staged_copy(src_at, dst_at, seg_len, stage):
  sr = stage.shape[0]
  nfull = seg_len // sr
  for t in range(nfull):
    pltpu.sync_copy(src_at(t * sr, sr), stage)
    pltpu.sync_copy(stage, dst_at(t * sr, sr))
  rem = seg_len - nfull * sr
  if rem:
    pltpu.sync_copy(src_at(nfull * sr, rem), stage.at[pl.ds(0, rem)])
    pltpu.sync_copy(stage.at[pl.ds(0, rem)], dst_at(nfull * sr, rem))


def _zero_init(zsrc_hbm, sh, stage, base, seg_len):
  sr = stage.shape[0]
  pltpu.sync_copy(zsrc_hbm.at[pl.ds(0, sr)], stage)
  for t in range(seg_len // sr):
    pltpu.sync_copy(stage, sh.at[pl.ds(base + t * sr, sr)])
  rem = seg_len % sr
  if rem:
    pltpu.sync_copy(stage.at[pl.ds(0, rem)], sh.at[pl.ds(base + (seg_len // sr) * sr, rem)])


def _unpack_src(pk, j, dst_ref, n=K):
  for t in range(n // L):
    v = pk[j, pl.ds(t * L, L)]
    dst_ref[pl.ds(t * L, L)] = v & 0x3FFF


def _unpack_dst(pk, j, dst_ref, n=K):
  for t in range(n // L):
    v = pk[j, pl.ds(t * L, L)]
    dst_ref[pl.ds(t * L, L)] = lax.shift_right_logical(v, 14)


KS = 32
RB = 8
SPC = K // KS
NSUB = CPWF * SPC
NQR = NSUB // RB


def _unpackKS(pk, row, part, is_ref, id_ref):
  for u in range(KS // L):
    v = pk[row, pl.ds(part * KS + u * L, L)]
    is_ref[pl.ds(u * L, L)] = v & 0x3FFF
    id_ref[pl.ds(u * L, L)] = lax.shift_right_logical(v, 14)


def _zfill(zsrc, sh, base, seg_len, stage, sems):
  pltpu.sync_copy(zsrc.at[pl.ds(0, stage.shape[0])], stage)
  segs = [(o, min(stage.shape[0], seg_len - o))
          for o in range(0, seg_len, stage.shape[0])]
  nf = len(sems)
  for i, (o, l) in enumerate(segs):
    m = i % nf
    if i >= nf:
      po, pl_ = segs[i - nf]
      pltpu.make_async_copy(stage.at[pl.ds(0, pl_)],
                            sh.at[pl.ds(base + po, pl_)], sems[m]).wait()
    pltpu.async_copy(stage.at[pl.ds(0, l)], sh.at[pl.ds(base + o, l)], sems[m])
  for i in range(max(0, len(segs) - nf), len(segs)):
    o, l = segs[i]
    pltpu.make_async_copy(stage.at[pl.ds(0, l)],
                          sh.at[pl.ds(base + o, l)], sems[i % nf]).wait()


def _sc_acc_body(g_hbm, pk_hbm, zrow_hbm, acc_out, pk, *rest):
  isb = list(rest[0:RB])
  idb = list(rest[RB:2 * RB])
  is_t, id_t = rest[2 * RB], rest[2 * RB + 1]
  rows = list(rest[2 * RB + 2:3 * RB + 2])
  acc_sh = rest[3 * RB + 2]
  gsem = list(rest[3 * RB + 3:4 * RB + 3])
  ssem = list(rest[4 * RB + 3:5 * RB + 3])

  c = lax.axis_index("c")
  s = lax.axis_index("s")
  w = c * NS + s
  base = s * ZSEG

  @pl.when(s < NS - 1)
  def _():
    _zfill(zrow_hbm, acc_sh, base, ZSEG, rows[0], gsem[:4])

  @pl.when(s == NS - 1)
  def _():
    _zfill(zrow_hbm, acc_sh, base, ZLAST, rows[0], gsem[:4])

  pltpu.sync_copy(pk_hbm.at[w], pk)
  plsc.subcore_barrier()

  def gstart(b):
    pltpu.async_copy(g_hbm.at[isb[b]], rows[b], gsem[b])

  def gwait(b):
    pltpu.make_async_copy(g_hbm.at[isb[b]], rows[b], gsem[b]).wait()

  def sstart(b):
    pltpu.async_copy(rows[b], acc_sh.at[idb[b]], ssem[b], add=True)

  def swait(b):
    pltpu.make_async_copy(rows[b], acc_sh.at[idb[b]], ssem[b]).wait()

  for b in range(RB):
    _unpackKS(pk, b // SPC, b % SPC, isb[b], idb[b])
    gstart(b)

  HB = RB // 2

  def body(q, carry):
    for b in range(RB):
      b2 = (b + HB) % RB
      def refill(bb=b2,
                 row=(2 * q + 1 + b // SPC) if b < HB else (2 * q + 2 + (b - HB) // SPC),
                 part=b % SPC):
        swait(bb)
        _unpackKS(pk, row, part, isb[bb], idb[bb])
        gstart(bb)

      gwait(b)
      sstart(b)
      if b < HB:
        pl.when(q > 0)(refill)
      else:
        pl.when(q < NQR - 1)(refill)
    return carry

  lax.fori_loop(0, NQR, body, 0)
  for b in range(RB):
    swait(b)

  def wb_async(seg_len):
    segs = [(o, min(KS, seg_len - o)) for o in range(0, seg_len, KS)]
    for i, (o, l) in enumerate(segs):
      b = i % RB
      if i >= RB:
        po, pl_ = segs[i - RB]
        pltpu.make_async_copy(
            rows[b].at[pl.ds(0, pl_)],
            acc_out.at[pl.ds(c * N + base + po, pl_)], ssem[b]).wait()
      pltpu.sync_copy(acc_sh.at[pl.ds(base + o, l)], rows[b].at[pl.ds(0, l)])
      pltpu.async_copy(rows[b].at[pl.ds(0, l)],
                       acc_out.at[pl.ds(c * N + base + o, l)], ssem[b])
    ntail = min(RB, len(segs))
    for i in range(len(segs) - ntail, len(segs)):
      o, l = segs[i]
      pltpu.make_async_copy(rows[i % RB].at[pl.ds(0, l)],
                            acc_out.at[pl.ds(c * N + base + o, l)],
                            ssem[i % RB]).wait()

  _unpack_src(pk, CPWF, is_t, n=TAIL)
  _unpack_dst(pk, CPWF, id_t, n=TAIL)
  pltpu.async_copy(g_hbm.at[is_t], rows[0].at[pl.ds(0, TAIL)], gsem[0]).wait()
  pltpu.sync_copy(rows[0].at[pl.ds(0, TAIL)], acc_sh.at[id_t], add=True)

  plsc.subcore_barrier()


  @pl.when(s < NS - 1)
  def _():
    wb_async(WSEG)

  @pl.when(s == NS - 1)
  def _():
    wb_async(WLAST)


_sc_scatter = pl.kernel(
    _sc_acc_body,
    mesh=_mesh,
    out_type=jax.ShapeDtypeStruct((NC * N, D), jnp.float32),
    scratch_types=(
        [pltpu.VMEM((CPW, K), jnp.int32)]
        + [pltpu.VMEM((KS,), jnp.int32) for _ in range(2 * RB)]
        + [pltpu.VMEM((TAIL,), jnp.int32) for _ in range(2)]
        + [pltpu.VMEM((KS, D), jnp.float32) for _ in range(RB)]
        + [pltpu.VMEM_SHARED((ROWS_SC, D), jnp.float32)]
        + [pltpu.SemaphoreType.DMA for _ in range(2 * RB)]
    ),
)


def _sc_cnt_body(pk_hbm, zrow_hbm, ones_hbm, cnt_out,
                 pk, id0, id1, id2, id3, id_t, ones_v, cbuf, cnt_sh,
                 ss0, ss1, ss2, ss3):
  c = lax.axis_index("c")
  s = lax.axis_index("s")
  w = c * NS + s
  base = s * ZSEG
  idb = [id0, id1, id2, id3]
  ssem = [ss0, ss1, ss2, ss3]

  @pl.when(s < NS - 1)
  def _():
    _zfill(zrow_hbm, cnt_sh, base, ZSEG, cbuf, ssem)

  @pl.when(s == NS - 1)
  def _():
    _zfill(zrow_hbm, cnt_sh, base, ZLAST, cbuf, ssem)

  pltpu.sync_copy(ones_hbm, ones_v)
  pltpu.sync_copy(pk_hbm.at[w], pk)
  plsc.subcore_barrier()

  def sstart(b):
    pltpu.async_copy(ones_v, cnt_sh.at[idb[b]], ssem[b], add=True)

  def swait(b):
    pltpu.make_async_copy(ones_v, cnt_sh.at[idb[b]], ssem[b]).wait()

  for b in range(4):
    _unpack_dst(pk, b, idb[b])
    sstart(b)

  def body(q, carry):
    for b in range(4):
      j = 4 * q + 4 + b
      swait(b)
      _unpack_dst(pk, j, idb[b])
      sstart(b)
    return carry

  lax.fori_loop(0, (CPWF - 6) // 4, body, 0)
  for b in range(2):
    swait(b)
    _unpack_dst(pk, CPWF - 2 + b, idb[b])
    sstart(b)
  for b in range(4):
    swait(b)

  _unpack_dst(pk, CPWF, id_t, n=TAIL)
  pltpu.sync_copy(ones_v.at[pl.ds(0, TAIL)], cnt_sh.at[id_t], add=True)

  plsc.subcore_barrier()

  @pl.when(s < NS - 1)
  def _():
    _staged_copy(lambda o, l: cnt_sh.at[pl.ds(base + o, l)],
                 lambda o, l: cnt_out.at[pl.ds(c * N + base + o, l)],
                 WSEG, cbuf)

  @pl.when(s == NS - 1)
  def _():
    _staged_copy(lambda o, l: cnt_sh.at[pl.ds(base + o, l)],
                 lambda o, l: cnt_out.at[pl.ds(c * N + base + o, l)],
                 WLAST, cbuf)


_sc_cnt = pl.kernel(
    _sc_cnt_body,
    mesh=_mesh,
    out_type=jax.ShapeDtypeStruct((NC * N, D), jnp.float32),
    scratch_types=[
        pltpu.VMEM((CPW, K), jnp.int32),
        pltpu.VMEM((K,), jnp.int32),
        pltpu.VMEM((K,), jnp.int32),
        pltpu.VMEM((K,), jnp.int32),
        pltpu.VMEM((K,), jnp.int32),
        pltpu.VMEM((TAIL,), jnp.int32),
        pltpu.VMEM((K, D), jnp.float32),
        pltpu.VMEM((K, D), jnp.float32),
        pltpu.VMEM_SHARED((ROWS_SC, D), jnp.float32),
        pltpu.SemaphoreType.DMA,
        pltpu.SemaphoreType.DMA,
        pltpu.SemaphoreType.DMA,
        pltpu.SemaphoreType.DMA,
    ],
)


def _tc_pre_body(x_ref, wl_ref, wr_ref, b_ref, g_ref, r_ref):
  h = x_ref[...]
  g_ref[...] = jnp.dot(h, wl_ref[...], preferred_element_type=jnp.float32)
  r_ref[...] = jnp.dot(h, wr_ref[...], preferred_element_type=jnp.float32) + b_ref[...]


_tc_pre = pl.pallas_call(
    _tc_pre_body,
    out_shape=[jax.ShapeDtypeStruct((N, D), jnp.float32),
               jax.ShapeDtypeStruct((N, D), jnp.float32)],
)


def _tc_mid_body(acc_ref, cnt_ref, rp_ref, wl_ref, wr_ref, b_ref,
                 g_ref, r_ref):
  acc = acc_ref[0] + acc_ref[1]
  cnt = cnt_ref[0] + cnt_ref[1]
  inv = 1.0 / jnp.maximum(cnt, 1.0)
  h = jnp.maximum(acc * inv + rp_ref[...], 0.0)
  g_ref[...] = jnp.dot(h, wl_ref[...], preferred_element_type=jnp.float32)
  r_ref[...] = jnp.dot(h, wr_ref[...], preferred_element_type=jnp.float32) + b_ref[...]


_tc_mid = pl.pallas_call(
    _tc_mid_body,
    out_shape=[jax.ShapeDtypeStruct((N, D), jnp.float32),
               jax.ShapeDtypeStruct((N, D), jnp.float32)],
)


def _tc_post_body(acc_ref, cnt_ref, rp_ref, out_ref):
  acc = acc_ref[0] + acc_ref[1]
  cnt = cnt_ref[0] + cnt_ref[1]
  inv = 1.0 / jnp.maximum(cnt, 1.0)
  out_ref[...] = acc * inv + rp_ref[...]


_tc_post = pl.pallas_call(
    _tc_post_body,
    out_shape=jax.ShapeDtypeStruct((N, D), jnp.float32),
)


def kernel(x, edge_index, Wl0, Wr0, b0, Wl1, Wr1, b1, Wl2, Wr2, b2):
  src = edge_index[0].astype(jnp.int32)
  dst = edge_index[1].astype(jnp.int32)
  ppw = CPW * K - EPW
  src_p = jnp.concatenate(
      [src.reshape(NW, EPW), jnp.zeros((NW, ppw), jnp.int32)], axis=1)
  dst_p = jnp.concatenate(
      [dst.reshape(NW, EPW), jnp.zeros((NW, ppw), jnp.int32)], axis=1)
  pk = (src_p | (dst_p << 14)).reshape(NW, CPW, K)
  zrow = jnp.zeros((K, D), jnp.float32)
  onesK = jnp.ones((K, D), jnp.float32)

  cnt = _sc_cnt(pk, zrow, onesK).reshape(NC, N, D)
  g0, r0 = _tc_pre(x, Wl0.T, Wr0.T, b0.reshape(1, D))
  acc0 = _sc_scatter(g0, pk, zrow).reshape(NC, N, D)
  g1, r1 = _tc_mid(acc0, cnt, r0, Wl1.T, Wr1.T, b1.reshape(1, D))
  acc1 = _sc_scatter(g1, pk, zrow).reshape(NC, N, D)
  g2, r2 = _tc_mid(acc1, cnt, r1, Wl2.T, Wr2.T, b2.reshape(1, D))
  acc2 = _sc_scatter(g2, pk, zrow).reshape(NC, N, D)
  return _tc_post(acc2, cnt, r2)


# speedup vs baseline: 1.0744x; 1.0011x over previous
import jax
import jax.numpy as jnp
from jax import lax
from jax.experimental import pallas as pl
from jax.experimental.pallas import tpu as pltpu
from jax.experimental.pallas import tpu_sc as plsc

N = 10000
D = 128
E = 320000
NC = 2
NS = 16
NW = NC * NS
K = 128
EPW = E // NW
CPWF = EPW // K
TAIL = EPW - CPWF * K
CPW = CPWF + 1
ROWS_SC = N
ZSEG = 624
ZLAST = ROWS_SC - (NS - 1) * ZSEG
WSEG = 624
WLAST = N - (NS - 1) * WSEG
L = 16

_mesh = plsc.VectorSubcoreMesh(core_axis_name="c", subcore_axis_name="s")


def _staged_copy(src_at, dst_at, seg_len, stage):
  sr = stage.shape[0]
  nfull = seg_len // sr
  for t in range(nfull):
    pltpu.sync_copy(src_at(t * sr, sr), stage)
    pltpu.sync_copy(stage, dst_at(t * sr, sr))
  rem = seg_len - nfull * sr
  if rem:
    pltpu.sync_copy(src_at(nfull * sr, rem), stage.at[pl.ds(0, rem)])
    pltpu.sync_copy(stage.at[pl.ds(0, rem)], dst_at(nfull * sr, rem))


def _unpack_src(pk, j, dst_ref, n=K):
  for t in range(n // L):
    v = pk[j, pl.ds(t * L, L)]
    dst_ref[pl.ds(t * L, L)] = v & 0x3FFF


def _unpack_dst(pk, j, dst_ref, n=K):
  for t in range(n // L):
    v = pk[j, pl.ds(t * L, L)]
    dst_ref[pl.ds(t * L, L)] = lax.shift_right_logical(v, 14)


KS = 32
RB = 8
SPC = K // KS
NSUB = CPWF * SPC
NQR = NSUB // RB


def _unpackKS(pk, row, part, is_ref, id_ref):
  for u in range(KS // L):
    v = pk[row, pl.ds(part * KS + u * L, L)]
    is_ref[pl.ds(u * L, L)] = v & 0x3FFF
    id_ref[pl.ds(u * L, L)] = lax.shift_right_logical(v, 14)


def _zfill(zsrc, sh, base, seg_len, stage, sems):
  pltpu.sync_copy(zsrc.at[pl.ds(0, stage.shape[0])], stage)
  segs = [(o, min(stage.shape[0], seg_len - o))
          for o in range(0, seg_len, stage.shape[0])]
  nf = len(sems)
  for i, (o, l) in enumerate(segs):
    m = i % nf
    if i >= nf:
      po, pl_ = segs[i - nf]
      pltpu.make_async_copy(stage.at[pl.ds(0, pl_)],
                            sh.at[pl.ds(base + po, pl_)], sems[m]).wait()
    pltpu.async_copy(stage.at[pl.ds(0, l)], sh.at[pl.ds(base + o, l)], sems[m])
  for i in range(max(0, len(segs) - nf), len(segs)):
    o, l = segs[i]
    pltpu.make_async_copy(stage.at[pl.ds(0, l)],
                          sh.at[pl.ds(base + o, l)], sems[i % nf]).wait()


def _sc_acc_body(g_hbm, pk_hbm, zrow_hbm, acc_out, pk, *rest):
  isb = list(rest[0:RB])
  idb = list(rest[RB:2 * RB])
  is_t, id_t = rest[2 * RB], rest[2 * RB + 1]
  rows = list(rest[2 * RB + 2:3 * RB + 2])
  acc_sh = rest[3 * RB + 2]
  gsem = list(rest[3 * RB + 3:4 * RB + 3])
  ssem = list(rest[4 * RB + 3:5 * RB + 3])

  c = lax.axis_index("c")
  s = lax.axis_index("s")
  w = c * NS + s
  base = s * ZSEG

  @pl.when(s < NS - 1)
  def _():
    _zfill(zrow_hbm, acc_sh, base, ZSEG, rows[0], gsem[:4])

  @pl.when(s == NS - 1)
  def _():
    _zfill(zrow_hbm, acc_sh, base, ZLAST, rows[0], gsem[:4])

  pltpu.sync_copy(pk_hbm.at[w], pk)
  plsc.subcore_barrier()

  def gstart(b):
    pltpu.async_copy(g_hbm.at[isb[b]], rows[b], gsem[b])

  def gwait(b):
    pltpu.make_async_copy(g_hbm.at[isb[b]], rows[b], gsem[b]).wait()

  def sstart(b):
    pltpu.async_copy(rows[b], acc_sh.at[idb[b]], ssem[b], add=True)

  def swait(b):
    pltpu.make_async_copy(rows[b], acc_sh.at[idb[b]], ssem[b]).wait()

  for b in range(RB):
    _unpackKS(pk, b // SPC, b % SPC, isb[b], idb[b])
    gstart(b)

  HB = RB // 2

  def body(q, carry):
    for b in range(RB):
      b2 = (b + HB) % RB
      def refill(bb=b2,
                 row=(2 * q + 1 + b // SPC) if b < HB else (2 * q + 2 + (b - HB) // SPC),
                 part=b % SPC):
        swait(bb)
        _unpackKS(pk, row, part, isb[bb], idb[bb])
        gstart(bb)

      gwait(b)
      sstart(b)
      if b < HB:
        pl.when(q > 0)(refill)
      else:
        pl.when(q < NQR - 1)(refill)
    return carry

  lax.fori_loop(0, NQR, body, 0)
  for b in range(RB):
    swait(b)

  def wb_async(seg_len):
    segs = [(o, min(KS, seg_len - o)) for o in range(0, seg_len, KS)]
    for i, (o, l) in enumerate(segs):
      b = i % RB
      if i >= RB:
        po, pl_ = segs[i - RB]
        pltpu.make_async_copy(
            rows[b].at[pl.ds(0, pl_)],
            acc_out.at[pl.ds(c * N + base + po, pl_)], ssem[b]).wait()
      pltpu.sync_copy(acc_sh.at[pl.ds(base + o, l)], rows[b].at[pl.ds(0, l)])
      pltpu.async_copy(rows[b].at[pl.ds(0, l)],
                       acc_out.at[pl.ds(c * N + base + o, l)], ssem[b])
    ntail = min(RB, len(segs))
    for i in range(len(segs) - ntail, len(segs)):
      o, l = segs[i]
      pltpu.make_async_copy(rows[i % RB].at[pl.ds(0, l)],
                            acc_out.at[pl.ds(c * N + base + o, l)],
                            ssem[i % RB]).wait()

  _unpack_src(pk, CPWF, is_t, n=TAIL)
  _unpack_dst(pk, CPWF, id_t, n=TAIL)
  pltpu.async_copy(g_hbm.at[is_t], rows[0].at[pl.ds(0, TAIL)], gsem[0]).wait()
  pltpu.sync_copy(rows[0].at[pl.ds(0, TAIL)], acc_sh.at[id_t], add=True)

  plsc.subcore_barrier()


  @pl.when(s < NS - 1)
  def _():
    wb_async(WSEG)

  @pl.when(s == NS - 1)
  def _():
    wb_async(WLAST)


_sc_scatter = pl.kernel(
    _sc_acc_body,
    mesh=_mesh,
    out_type=jax.ShapeDtypeStruct((NC * N, D), jnp.float32),
    scratch_types=(
        [pltpu.VMEM((CPW, K), jnp.int32)]
        + [pltpu.VMEM((KS,), jnp.int32) for _ in range(2 * RB)]
        + [pltpu.VMEM((TAIL,), jnp.int32) for _ in range(2)]
        + [pltpu.VMEM((KS, D), jnp.float32) for _ in range(RB)]
        + [pltpu.VMEM_SHARED((ROWS_SC, D), jnp.float32)]
        + [pltpu.SemaphoreType.DMA for _ in range(2 * RB)]
    ),
)


def _sc_cnt_body(pk_hbm, zrow_hbm, ones_hbm, cnt_out,
                 pk, id0, id1, id2, id3, id_t, ones_v, cbuf, cnt_sh,
                 ss0, ss1, ss2, ss3):
  c = lax.axis_index("c")
  s = lax.axis_index("s")
  w = c * NS + s
  base = s * ZSEG
  idb = [id0, id1, id2, id3]
  ssem = [ss0, ss1, ss2, ss3]

  @pl.when(s < NS - 1)
  def _():
    _zfill(zrow_hbm, cnt_sh, base, ZSEG, cbuf, ssem)

  @pl.when(s == NS - 1)
  def _():
    _zfill(zrow_hbm, cnt_sh, base, ZLAST, cbuf, ssem)

  pltpu.sync_copy(ones_hbm, ones_v)
  pltpu.sync_copy(pk_hbm.at[w], pk)
  plsc.subcore_barrier()

  def sstart(b):
    pltpu.async_copy(ones_v, cnt_sh.at[idb[b]], ssem[b], add=True)

  def swait(b):
    pltpu.make_async_copy(ones_v, cnt_sh.at[idb[b]], ssem[b]).wait()

  for b in range(4):
    _unpack_dst(pk, b, idb[b])
    sstart(b)

  def body(q, carry):
    for b in range(4):
      j = 4 * q + 4 + b
      swait(b)
      _unpack_dst(pk, j, idb[b])
      sstart(b)
    return carry

  lax.fori_loop(0, (CPWF - 6) // 4, body, 0)
  for b in range(2):
    swait(b)
    _unpack_dst(pk, CPWF - 2 + b, idb[b])
    sstart(b)
  for b in range(4):
    swait(b)

  _unpack_dst(pk, CPWF, id_t, n=TAIL)
  pltpu.sync_copy(ones_v.at[pl.ds(0, TAIL)], cnt_sh.at[id_t], add=True)

  plsc.subcore_barrier()

  @pl.when(s < NS - 1)
  def _():
    _staged_copy(lambda o, l: cnt_sh.at[pl.ds(base + o, l)],
                 lambda o, l: cnt_out.at[pl.ds(c * N + base + o, l)],
                 WSEG, cbuf)

  @pl.when(s == NS - 1)
  def _():
    _staged_copy(lambda o, l: cnt_sh.at[pl.ds(base + o, l)],
                 lambda o, l: cnt_out.at[pl.ds(c * N + base + o, l)],
                 WLAST, cbuf)


_sc_cnt = pl.kernel(
    _sc_cnt_body,
    mesh=_mesh,
    out_type=jax.ShapeDtypeStruct((NC * N, D), jnp.float32),
    scratch_types=[
        pltpu.VMEM((CPW, K), jnp.int32),
        pltpu.VMEM((K,), jnp.int32),
        pltpu.VMEM((K,), jnp.int32),
        pltpu.VMEM((K,), jnp.int32),
        pltpu.VMEM((K,), jnp.int32),
        pltpu.VMEM((TAIL,), jnp.int32),
        pltpu.VMEM((K, D), jnp.float32),
        pltpu.VMEM((K, D), jnp.float32),
        pltpu.VMEM_SHARED((ROWS_SC, D), jnp.float32),
        pltpu.SemaphoreType.DMA,
        pltpu.SemaphoreType.DMA,
        pltpu.SemaphoreType.DMA,
        pltpu.SemaphoreType.DMA,
    ],
)


def _tc_pre_body(x_ref, wl_ref, wr_ref, b_ref, g_ref, r_ref):
  h = x_ref[...]
  g_ref[...] = jnp.dot(h, wl_ref[...], preferred_element_type=jnp.float32)
  r_ref[...] = jnp.dot(h, wr_ref[...], preferred_element_type=jnp.float32) + b_ref[...]


_tc_pre = pl.pallas_call(
    _tc_pre_body,
    out_shape=[jax.ShapeDtypeStruct((N, D), jnp.float32),
               jax.ShapeDtypeStruct((N, D), jnp.float32)],
)


def _tc_mid_body(acc_ref, cnt_ref, rp_ref, wl_ref, wr_ref, b_ref,
                 g_ref, r_ref):
  acc = acc_ref[0] + acc_ref[1]
  cnt = cnt_ref[0] + cnt_ref[1]
  inv = 1.0 / jnp.maximum(cnt, 1.0)
  h = jnp.maximum(acc * inv + rp_ref[...], 0.0)
  g_ref[...] = jnp.dot(h, wl_ref[...], preferred_element_type=jnp.float32)
  r_ref[...] = jnp.dot(h, wr_ref[...], preferred_element_type=jnp.float32) + b_ref[...]


_tc_mid = pl.pallas_call(
    _tc_mid_body,
    out_shape=[jax.ShapeDtypeStruct((N, D), jnp.float32),
               jax.ShapeDtypeStruct((N, D), jnp.float32)],
)


def _tc_post_body(acc_ref, cnt_ref, rp_ref, out_ref):
  acc = acc_ref[0] + acc_ref[1]
  cnt = cnt_ref[0] + cnt_ref[1]
  inv = 1.0 / jnp.maximum(cnt, 1.0)
  out_ref[...] = acc * inv + rp_ref[...]


_tc_post = pl.pallas_call(
    _tc_post_body,
    out_shape=jax.ShapeDtypeStruct((N, D), jnp.float32),
)


def kernel(x, edge_index, Wl0, Wr0, b0, Wl1, Wr1, b1, Wl2, Wr2, b2):
  src = edge_index[0].astype(jnp.int32)
  dst = edge_index[1].astype(jnp.int32)
  ppw = CPW * K - EPW
  src_p = jnp.concatenate(
      [src.reshape(NW, EPW), jnp.zeros((NW, ppw), jnp.int32)], axis=1)
  dst_p = jnp.concatenate(
      [dst.reshape(NW, EPW), jnp.zeros((NW, ppw), jnp.int32)], axis=1)
  pk = (src_p | (dst_p << 14)).reshape(NW, CPW, K)
  zrow = jnp.zeros((K, D), jnp.float32)
  onesK = jnp.ones((K, D), jnp.float32)

  cnt = _sc_cnt(pk, zrow, onesK).reshape(NC, N, D)
  g0, r0 = _tc_pre(x, Wl0.T, Wr0.T, b0.reshape(1, D))
  acc0 = _sc_scatter(g0, pk, zrow).reshape(NC, N, D)
  g1, r1 = _tc_mid(acc0, cnt, r0, Wl1.T, Wr1.T, b1.reshape(1, D))
  acc1 = _sc_scatter(g1, pk, zrow).reshape(NC, N, D)
  g2, r2 = _tc_mid(acc1, cnt, r1, Wl2.T, Wr2.T, b2.reshape(1, D))
  acc2 = _sc_scatter(g2, pk, zrow).reshape(NC, N, D)
  return _tc_post(acc2, cnt, r2)
